# baseline (speedup 1.0000x reference)
import functools

import jax
import jax.numpy as jnp
from jax import lax
from jax.experimental import pallas as pl
from jax.experimental.pallas import tpu as pltpu
from jax.experimental.pallas import tpu_sc as plsc

B = 4096
L = 200
F = 32
NC = 2
NS = 16
NW = NC * NS
BPW = B // NW
HALF = F // 2
NBUF = 4
PHN = (72, 64, 64)
PHOFF = (0, 72, 136)
NPH = 3
NSTEP = BPW * NPH

_GATHER_DNUMS = lax.GatherDimensionNumbers(
    offset_dims=(), collapsed_slice_dims=(0,), start_index_map=(0,))


def _sc_body(cx0_hbm, cx1_hbm, cx2_hbm, nx0_hbm, nx1_hbm, nx2_hbm,
             t0_hbm, t1_hbm, t2_hbm, out_hbm,
             crow_v, cqo_v, nrow_v, nqo_v, b0_v, b1_v, b2_v, b3_v,
             nrows_v, pacc_v, out_v,
             sem0, sem1, sem2, sem3, nsem):
    wid = lax.axis_index("s") * NC + lax.axis_index("c")

    zero = jnp.zeros((HALF,), jnp.float32)
    lanes = lax.iota(jnp.int32, HALF)
    bufs = (b0_v, b1_v, b2_v, b3_v)
    sems = (sem0, sem1, sem2, sem3)

    def zpacc(i, _):
        pacc_v[pl.ds(i * HALF, HALF)] = zero
        return 0

    lax.fori_loop(0, BPW, zpacc, 0)

    def issue(tab_hbm, step_b, ph, buf, sm):
        src = tab_hbm.at[crow_v.at[pl.ds(step_b * L + PHOFF[ph], PHN[ph])]]
        pltpu.async_copy(src, buf.at[pl.ds(0, PHN[ph])], sm)

    def drain(tab_hbm, ph, buf, sm):
        src = tab_hbm.at[crow_v.at[pl.ds(0, PHN[ph])]]
        pltpu.make_async_copy(src, buf.at[pl.ds(0, PHN[ph])], sm).wait()

    def accum(fb, n, buf, carry):
        def blk(k, c):
            a0, a1 = c
            l0 = k * 16
            qv = cqo_v[pl.ds(fb + l0, HALF)]
            for u2 in range(16):
                qo = pl.multiple_of(qv[u2], 32)
                a0 = a0 + buf[l0 + u2, pl.ds(qo, HALF)]
                a1 = a1 + buf[l0 + u2, pl.ds(qo + HALF, HALF)]
            return a0, a1

        c = lax.fori_loop(0, n // 16, blk, carry)
        if n % 16:
            l0 = n - 8
            qv = cqo_v[pl.ds(fb + n - 16, HALF)]
            a0, a1 = c
            for j in range(8):
                qo = pl.multiple_of(qv[8 + j], 32)
                a0 = a0 + buf[l0 + j, pl.ds(qo, HALF)]
                a1 = a1 + buf[l0 + j, pl.ds(qo + HALF, HALF)]
            c = (a0, a1)
        return c

    def finalize(b):
        def fn(a0, a1):
            nb = pl.multiple_of((b // 8) * 8, 8)
            qv = nqo_v[pl.ds(nb, HALF)]
            qrep = jnp.where(lanes == (b - nb), qv, 0)
            for sh in (8, 4, 2, 1):
                qrep = qrep + lax.gather(
                    qrep, (lanes ^ sh)[:, None], _GATHER_DNUMS, (1,),
                    mode=lax.GatherScatterMode.PROMISE_IN_BOUNDS)
            n0 = zero
            n1 = zero
            for q in range(4):
                m = qrep == (q * 32)
                n0 = jnp.where(m, nrows_v[b, pl.ds(q * 32, HALF)], n0)
                n1 = jnp.where(m, nrows_v[b, pl.ds(q * 32 + HALF, HALF)], n1)
            p = a0 * n0 + a1 * n1
            off = b * HALF
            pacc_v[pl.ds(off, HALF)] = pacc_v[pl.ds(off, HALF)] + p
        return fn

    for cx_hbm, nx_hbm, tab_hbm in ((cx0_hbm, nx0_hbm, t0_hbm),
                                    (cx1_hbm, nx1_hbm, t1_hbm),
                                    (cx2_hbm, nx2_hbm, t2_hbm)):
        pltpu.sync_copy(cx_hbm.at[wid, pl.ds(0, BPW * L)], crow_v)
        pltpu.sync_copy(cx_hbm.at[wid, pl.ds(BPW * L, BPW * L)], cqo_v)
        pltpu.sync_copy(nx_hbm.at[wid, pl.ds(0, BPW)], nrow_v)
        pltpu.sync_copy(nx_hbm.at[wid, pl.ds(BPW, BPW)], nqo_v)
        pltpu.async_copy(tab_hbm.at[nrow_v], nrows_v, nsem).wait()

        for u in range(NBUF - 1):
            issue(tab_hbm, 0, u, bufs[u], sems[u])

        def g_body(g, _):
            carry = (zero, zero)
            for u in range(4 * NPH):
                step = 12 * g + u
                b = 4 * g + u // NPH
                ph = u % NPH
                s_next = step + NBUF - 1
                un = u + NBUF - 1
                ph_next = un % NPH

                @pl.when(s_next < NSTEP)
                def _():
                    issue(tab_hbm, s_next // NPH, ph_next,
                          bufs[un % NBUF], sems[un % NBUF])

                drain(tab_hbm, ph, bufs[u % NBUF], sems[u % NBUF])
                if ph == 0:
                    carry = (zero, zero)
                carry = accum(b * L + PHOFF[ph], PHN[ph],
                              bufs[u % NBUF], carry)
                if ph == NPH - 1:
                    finalize(b)(*carry)
            return 0

        lax.fori_loop(0, BPW // 4, g_body, 0)

    inv_l = jnp.float32(1.0 / L)

    def out_chunk(g, _):
        def out_lane(j, out_acc):
            p = pacc_v[pl.ds((g * HALF + j) * HALF, HALF)]
            for sh in (8, 4, 2, 1):
                p = p + lax.gather(
                    p, (lanes ^ sh)[:, None], _GATHER_DNUMS, (1,),
                    mode=lax.GatherScatterMode.PROMISE_IN_BOUNDS)
            return jnp.where(lanes == j, p, out_acc)

        out_acc = lax.fori_loop(0, HALF, out_lane, zero)
        out_v[pl.ds(g * HALF, HALF)] = out_acc * inv_l
        return 0

    lax.fori_loop(0, BPW // HALF, out_chunk, 0)
    pltpu.sync_copy(out_v, out_hbm.at[wid])


_spotify_sc = functools.partial(
    pl.kernel,
    mesh=plsc.VectorSubcoreMesh(core_axis_name="c", subcore_axis_name="s"),
    out_type=jax.ShapeDtypeStruct((NW, BPW), jnp.float32),
    compiler_params=pltpu.CompilerParams(use_tc_tiling_on_sc=True),
    scratch_types=[
        pltpu.VMEM((BPW * L,), jnp.int32),
        pltpu.VMEM((BPW * L,), jnp.int32),
        pltpu.VMEM((BPW,), jnp.int32),
        pltpu.VMEM((BPW,), jnp.int32),
        pltpu.VMEM((PHN[0], 128), jnp.float32),
        pltpu.VMEM((PHN[0], 128), jnp.float32),
        pltpu.VMEM((PHN[0], 128), jnp.float32),
        pltpu.VMEM((PHN[0], 128), jnp.float32),
        pltpu.VMEM((BPW, 128), jnp.float32),
        pltpu.VMEM((BPW * HALF,), jnp.float32),
        pltpu.VMEM((BPW,), jnp.float32),
        pltpu.SemaphoreType.DMA,
        pltpu.SemaphoreType.DMA,
        pltpu.SemaphoreType.DMA,
        pltpu.SemaphoreType.DMA,
        pltpu.SemaphoreType.DMA,
    ],
)(_sc_body)


def _pack_ctx(ctx):
    flat = ctx.astype(jnp.int32).reshape(NW, BPW * L)
    return jnp.concatenate([flat >> 2, (flat & 3) << 5], axis=1)


def _pack_next(nxt):
    flat = nxt.astype(jnp.int32).reshape(NW, BPW)
    return jnp.concatenate([flat >> 2, (flat & 3) << 5], axis=1)


def kernel(track_context, artist_context, album_context,
           next_track, next_artist, next_album,
           track_table, artist_table, album_table):
    cx0 = _pack_ctx(track_context)
    cx1 = _pack_ctx(artist_context)
    cx2 = _pack_ctx(album_context)
    nx0 = _pack_next(next_track)
    nx1 = _pack_next(next_artist)
    nx2 = _pack_next(next_album)
    t0 = track_table.reshape(-1, 128)
    t1 = artist_table.reshape(-1, 128)
    t2 = album_table.reshape(-1, 128)
    out = _spotify_sc(cx0, cx1, cx2, nx0, nx1, nx2, t0, t1, t2)
    return out.reshape(B)


# speedup vs baseline: 1.8215x; 1.8215x over previous
import functools

import jax
import jax.numpy as jnp
from jax import lax
from jax.experimental import pallas as pl
from jax.experimental.pallas import tpu as pltpu
from jax.experimental.pallas import tpu_sc as plsc

B = 4096
L = 200
F = 32
NC = 2
NS = 16
NW = NC * NS
BPW = B // NW
C1 = 128
C2 = L - C1
HALF = F // 2
NBUF = 4
CH = 8192

_GATHER_DNUMS = lax.GatherDimensionNumbers(
    offset_dims=(), collapsed_slice_dims=(0,), start_index_map=(0,))


QT = CH // 4


def _tp_body(x_ref, o_ref):
    y = x_ref[...].T
    o_ref[...] = jnp.concatenate(
        [y[a * QT:(a + 1) * QT, :] for a in range(4)], axis=1)


def _linearize_table(table):
    v = table.shape[0]
    tab_t = table.T
    nblk = (v + CH - 1) // CH
    t4 = pl.pallas_call(
        _tp_body,
        grid=(nblk,),
        in_specs=[pl.BlockSpec((F, CH), lambda c: (0, c))],
        out_specs=pl.BlockSpec((QT, 128), lambda c: (c, 0)),
        out_shape=jax.ShapeDtypeStruct((nblk * QT, 128), jnp.float32),
    )(tab_t)
    return t4.reshape(nblk * CH, F)


def _perm_ids(ids):
    i = ids.astype(jnp.int32)
    c = i // CH
    l = i % CH
    return (c * QT + l % QT) * 4 + l // QT


def _sc_body(tc_hbm, ac_hbm, alc_hbm, nt_hbm, na_hbm, nal_hbm,
             ttab_hbm, atab_hbm, altab_hbm, out_hbm,
             ctx_v, nidx_v, r0_v, r1_v, r2_v, r3_v, nrows_v, pacc_v, out_v,
             sem0, sem1, sem2, sem3, nsem):
    wid = lax.axis_index("s") * NC + lax.axis_index("c")
    base = wid * BPW

    zero = jnp.zeros((HALF,), jnp.float32)
    lanes = lax.iota(jnp.int32, HALF)
    bufs = (r0_v, r1_v, r2_v, r3_v)
    sems = (sem0, sem1, sem2, sem3)

    def zpacc(i, _):
        pacc_v[pl.ds(i * HALF, HALF)] = zero
        return 0

    lax.fori_loop(0, BPW, zpacc, 0)

    def issue(tab_hbm, b, buf, sm):
        pltpu.async_copy(tab_hbm.at[ctx_v.at[b, pl.ds(0, C1)]],
                         buf.at[pl.ds(0, C1)], sm)
        pltpu.async_copy(tab_hbm.at[ctx_v.at[b, pl.ds(C1, C2)]],
                         buf.at[pl.ds(C1, C2)], sm)

    def drain(tab_hbm, buf, sm):
        pltpu.make_async_copy(tab_hbm.at[ctx_v.at[0, pl.ds(0, C1)]],
                              buf.at[pl.ds(0, C1)], sm).wait()
        pltpu.make_async_copy(tab_hbm.at[ctx_v.at[0, pl.ds(C1, C2)]],
                              buf.at[pl.ds(C1, C2)], sm).wait()

    def accum(b, buf):
        def row_body(r, carry):
            a0, a1 = carry
            l0 = r * 8
            for u in range(8):
                a0 = a0 + buf[l0 + u, pl.ds(0, HALF)]
                a1 = a1 + buf[l0 + u, pl.ds(HALF, HALF)]
            return a0, a1

        a0, a1 = lax.fori_loop(0, L // 8, row_body, (zero, zero))
        p = (a0 * nrows_v[b, pl.ds(0, HALF)]
             + a1 * nrows_v[b, pl.ds(HALF, HALF)])
        off = b * HALF
        pacc_v[pl.ds(off, HALF)] = pacc_v[pl.ds(off, HALF)] + p

    for ctx_hbm, next_hbm, tab_hbm in ((tc_hbm, nt_hbm, ttab_hbm),
                                       (ac_hbm, na_hbm, atab_hbm),
                                       (alc_hbm, nal_hbm, altab_hbm)):
        pltpu.sync_copy(ctx_hbm.at[pl.ds(base, BPW)], ctx_v)
        pltpu.sync_copy(next_hbm.at[pl.ds(base, BPW)], nidx_v)
        pltpu.async_copy(tab_hbm.at[nidx_v], nrows_v, nsem).wait()

        for u in range(NBUF - 1):
            issue(tab_hbm, u, bufs[u], sems[u])

        def g_body(g, _):
            for u in range(NBUF):
                b = NBUF * g + u
                b_next = b + NBUF - 1

                @pl.when(b_next < BPW)
                def _():
                    issue(tab_hbm, b_next, bufs[(u + NBUF - 1) % NBUF],
                          sems[(u + NBUF - 1) % NBUF])

                drain(tab_hbm, bufs[u], sems[u])
                accum(b, bufs[u])
            return 0

        lax.fori_loop(0, BPW // NBUF, g_body, 0)

    inv_l = jnp.float32(1.0 / L)

    def out_chunk(g, _):
        def out_lane(j, out_acc):
            p = pacc_v[pl.ds((g * HALF + j) * HALF, HALF)]
            for sh in (8, 4, 2, 1):
                p = p + lax.gather(
                    p, (lanes ^ sh)[:, None], _GATHER_DNUMS, (1,),
                    mode=lax.GatherScatterMode.PROMISE_IN_BOUNDS)
            return jnp.where(lanes == j, p, out_acc)

        out_acc = lax.fori_loop(0, HALF, out_lane, zero)
        out_v[pl.ds(g * HALF, HALF)] = out_acc * inv_l
        return 0

    lax.fori_loop(0, BPW // HALF, out_chunk, 0)
    pltpu.sync_copy(out_v, out_hbm.at[pl.ds(base, BPW)])


_spotify_sc = functools.partial(
    pl.kernel,
    mesh=plsc.VectorSubcoreMesh(core_axis_name="c", subcore_axis_name="s"),
    out_type=jax.ShapeDtypeStruct((B,), jnp.float32),
    compiler_params=pltpu.CompilerParams(use_tc_tiling_on_sc=False),
    scratch_types=[
        pltpu.VMEM((BPW, L), jnp.int32),
        pltpu.VMEM((BPW,), jnp.int32),
        pltpu.VMEM((L, F), jnp.float32),
        pltpu.VMEM((L, F), jnp.float32),
        pltpu.VMEM((L, F), jnp.float32),
        pltpu.VMEM((L, F), jnp.float32),
        pltpu.VMEM((BPW, F), jnp.float32),
        pltpu.VMEM((BPW * HALF,), jnp.float32),
        pltpu.VMEM((BPW,), jnp.float32),
        pltpu.SemaphoreType.DMA,
        pltpu.SemaphoreType.DMA,
        pltpu.SemaphoreType.DMA,
        pltpu.SemaphoreType.DMA,
        pltpu.SemaphoreType.DMA,
    ],
)(_sc_body)


def kernel(track_context, artist_context, album_context,
           next_track, next_artist, next_album,
           track_table, artist_table, album_table):
    tc = _perm_ids(track_context)
    ac = _perm_ids(artist_context)
    alc = _perm_ids(album_context)
    nt = _perm_ids(next_track.reshape(-1))
    na = _perm_ids(next_artist.reshape(-1))
    nal = _perm_ids(next_album.reshape(-1))
    t0 = _linearize_table(track_table)
    t1 = _linearize_table(artist_table)
    t2 = _linearize_table(album_table)
    return _spotify_sc(tc, ac, alc, nt, na, nal, t0, t1, t2)


# speedup vs baseline: 2.6099x; 1.4329x over previous
import functools

import jax
import jax.numpy as jnp
from jax import lax
from jax.experimental import pallas as pl
from jax.experimental.pallas import tpu as pltpu
from jax.experimental.pallas import tpu_sc as plsc

B = 4096
L = 200
F = 32
NC = 2
NS = 16
NW = NC * NS
BPW = B // NW
C1 = 128
C2 = L - C1
HALF = F // 2
NBUF = 4
CH = 16384

_GATHER_DNUMS = lax.GatherDimensionNumbers(
    offset_dims=(), collapsed_slice_dims=(0,), start_index_map=(0,))


QT = CH // 4


def _tp_body(x_ref, o_ref):
    rows = lax.broadcasted_iota(jnp.int32, (F, 128), 0)
    cols = lax.broadcasted_iota(jnp.int32, (F, 128), 1)
    acc = None
    for a in range(4):
        xa = x_ref[:, a * QT:(a + 1) * QT]
        ea = (rows + F * a == cols).astype(jnp.float32)
        d = lax.dot_general(xa, ea, (((0,), (0,)), ((), ())),
                            preferred_element_type=jnp.float32)
        acc = d if acc is None else acc + d
    o_ref[...] = acc


def _linearize_table(table):
    v = table.shape[0]
    tab_t = table.T
    nblk = (v + CH - 1) // CH
    t4 = pl.pallas_call(
        _tp_body,
        grid=(nblk,),
        in_specs=[pl.BlockSpec((F, CH), lambda c: (0, c))],
        out_specs=pl.BlockSpec((QT, 128), lambda c: (c, 0)),
        out_shape=jax.ShapeDtypeStruct((nblk * QT, 128), jnp.float32),
        compiler_params=pltpu.CompilerParams(fuse_transposed_lhs_in_matmul=True),
    )(tab_t)
    return t4.reshape(nblk * CH, F)


def _perm_ids(ids):
    i = ids.astype(jnp.int32)
    c = i // CH
    l = i % CH
    return (c * QT + l % QT) * 4 + l // QT


def _sc_body(tc_hbm, ac_hbm, alc_hbm, nt_hbm, na_hbm, nal_hbm,
             ttab_hbm, atab_hbm, altab_hbm, out_hbm,
             ctx_v, nidx_v, r0_v, r1_v, r2_v, r3_v, nrows_v, pacc_v, out_v,
             sem0, sem1, sem2, sem3, nsem):
    wid = lax.axis_index("s") * NC + lax.axis_index("c")
    base = wid * BPW

    zero = jnp.zeros((HALF,), jnp.float32)
    lanes = lax.iota(jnp.int32, HALF)
    bufs = (r0_v, r1_v, r2_v, r3_v)
    sems = (sem0, sem1, sem2, sem3)

    def zpacc(i, _):
        pacc_v[pl.ds(i * HALF, HALF)] = zero
        return 0

    lax.fori_loop(0, BPW, zpacc, 0)

    def issue(tab_hbm, b, buf, sm):
        pltpu.async_copy(tab_hbm.at[ctx_v.at[b, pl.ds(0, C1)]],
                         buf.at[pl.ds(0, C1)], sm)
        pltpu.async_copy(tab_hbm.at[ctx_v.at[b, pl.ds(C1, C2)]],
                         buf.at[pl.ds(C1, C2)], sm)

    def drain(tab_hbm, buf, sm):
        pltpu.make_async_copy(tab_hbm.at[ctx_v.at[0, pl.ds(0, C1)]],
                              buf.at[pl.ds(0, C1)], sm).wait()
        pltpu.make_async_copy(tab_hbm.at[ctx_v.at[0, pl.ds(C1, C2)]],
                              buf.at[pl.ds(C1, C2)], sm).wait()

    def accum(b, buf):
        def row_body(r, carry):
            a0, a1 = carry
            l0 = r * 8
            for u in range(8):
                a0 = a0 + buf[l0 + u, pl.ds(0, HALF)]
                a1 = a1 + buf[l0 + u, pl.ds(HALF, HALF)]
            return a0, a1

        a0, a1 = lax.fori_loop(0, L // 8, row_body, (zero, zero))
        p = (a0 * nrows_v[b, pl.ds(0, HALF)]
             + a1 * nrows_v[b, pl.ds(HALF, HALF)])
        off = b * HALF
        pacc_v[pl.ds(off, HALF)] = pacc_v[pl.ds(off, HALF)] + p

    for ctx_hbm, next_hbm, tab_hbm in ((tc_hbm, nt_hbm, ttab_hbm),
                                       (ac_hbm, na_hbm, atab_hbm),
                                       (alc_hbm, nal_hbm, altab_hbm)):
        pltpu.sync_copy(ctx_hbm.at[pl.ds(base, BPW)], ctx_v)
        pltpu.sync_copy(next_hbm.at[pl.ds(base, BPW)], nidx_v)
        pltpu.async_copy(tab_hbm.at[nidx_v], nrows_v, nsem).wait()

        for u in range(NBUF - 1):
            issue(tab_hbm, u, bufs[u], sems[u])

        def g_body(g, _):
            for u in range(NBUF):
                b = NBUF * g + u
                b_next = b + NBUF - 1

                @pl.when(b_next < BPW)
                def _():
                    issue(tab_hbm, b_next, bufs[(u + NBUF - 1) % NBUF],
                          sems[(u + NBUF - 1) % NBUF])

                drain(tab_hbm, bufs[u], sems[u])
                accum(b, bufs[u])
            return 0

        lax.fori_loop(0, BPW // NBUF, g_body, 0)

    inv_l = jnp.float32(1.0 / L)

    def out_chunk(g, _):
        def out_lane(j, out_acc):
            p = pacc_v[pl.ds((g * HALF + j) * HALF, HALF)]
            for sh in (8, 4, 2, 1):
                p = p + lax.gather(
                    p, (lanes ^ sh)[:, None], _GATHER_DNUMS, (1,),
                    mode=lax.GatherScatterMode.PROMISE_IN_BOUNDS)
            return jnp.where(lanes == j, p, out_acc)

        out_acc = lax.fori_loop(0, HALF, out_lane, zero)
        out_v[pl.ds(g * HALF, HALF)] = out_acc * inv_l
        return 0

    lax.fori_loop(0, BPW // HALF, out_chunk, 0)
    pltpu.sync_copy(out_v, out_hbm.at[pl.ds(base, BPW)])


_spotify_sc = functools.partial(
    pl.kernel,
    mesh=plsc.VectorSubcoreMesh(core_axis_name="c", subcore_axis_name="s"),
    out_type=jax.ShapeDtypeStruct((B,), jnp.float32),
    compiler_params=pltpu.CompilerParams(use_tc_tiling_on_sc=False),
    scratch_types=[
        pltpu.VMEM((BPW, L), jnp.int32),
        pltpu.VMEM((BPW,), jnp.int32),
        pltpu.VMEM((L, F), jnp.float32),
        pltpu.VMEM((L, F), jnp.float32),
        pltpu.VMEM((L, F), jnp.float32),
        pltpu.VMEM((L, F), jnp.float32),
        pltpu.VMEM((BPW, F), jnp.float32),
        pltpu.VMEM((BPW * HALF,), jnp.float32),
        pltpu.VMEM((BPW,), jnp.float32),
        pltpu.SemaphoreType.DMA,
        pltpu.SemaphoreType.DMA,
        pltpu.SemaphoreType.DMA,
        pltpu.SemaphoreType.DMA,
        pltpu.SemaphoreType.DMA,
    ],
)(_sc_body)


def kernel(track_context, artist_context, album_context,
           next_track, next_artist, next_album,
           track_table, artist_table, album_table):
    tc = _perm_ids(track_context)
    ac = _perm_ids(artist_context)
    alc = _perm_ids(album_context)
    nt = _perm_ids(next_track.reshape(-1))
    na = _perm_ids(next_artist.reshape(-1))
    nal = _perm_ids(next_album.reshape(-1))
    t0 = _linearize_table(track_table)
    t1 = _linearize_table(artist_table)
    t2 = _linearize_table(album_table)
    return _spotify_sc(tc, ac, alc, nt, na, nal, t0, t1, t2)


# speedup vs baseline: 3.8508x; 1.4754x over previous
import functools

import jax
import jax.numpy as jnp
from jax import lax
from jax.experimental import pallas as pl
from jax.experimental.pallas import tpu as pltpu
from jax.experimental.pallas import tpu_sc as plsc

B = 4096
L = 200
F = 32
NC = 2
NS = 16
NW = NC * NS
BPW = B // NW
C1 = 128
C2 = L - C1
HALF = F // 2
NBUF = 4
CH = 16384

_GATHER_DNUMS = lax.GatherDimensionNumbers(
    offset_dims=(), collapsed_slice_dims=(0,), start_index_map=(0,))


QT = CH // 4


def _tp_body(x_ref, o_ref):
    xs = jnp.concatenate(
        [x_ref[:, a * QT:(a + 1) * QT] for a in range(4)], axis=0)
    o_ref[...] = xs.T


def _linearize_table(table):
    v = table.shape[0]
    tab_t = table.T
    nblk = (v + CH - 1) // CH
    t4 = pl.pallas_call(
        _tp_body,
        grid=(nblk,),
        in_specs=[pl.BlockSpec((F, CH), lambda c: (0, c))],
        out_specs=pl.BlockSpec((QT, 128), lambda c: (c, 0)),
        out_shape=jax.ShapeDtypeStruct((nblk * QT, 128), jnp.float32),
        compiler_params=pltpu.CompilerParams(fuse_transposed_lhs_in_matmul=True),
    )(tab_t)
    return t4.reshape(nblk * CH, F)


def _perm_ids(ids):
    i = ids.astype(jnp.int32)
    c = i // CH
    l = i % CH
    return (c * QT + l % QT) * 4 + l // QT


def _sc_body(ctx_hbm, next_hbm, tab_hbm, out_hbm,
             ctx_v, nidx_v, r0_v, r1_v, r2_v, r3_v, nrows_v, pacc_v, out_v,
             sem0, sem1, sem2, sem3, nsem):
    wid = lax.axis_index("s") * NC + lax.axis_index("c")
    base = wid * BPW

    zero = jnp.zeros((HALF,), jnp.float32)
    lanes = lax.iota(jnp.int32, HALF)
    bufs = (r0_v, r1_v, r2_v, r3_v)
    sems = (sem0, sem1, sem2, sem3)

    def zpacc(i, _):
        pacc_v[pl.ds(i * HALF, HALF)] = zero
        return 0

    lax.fori_loop(0, BPW, zpacc, 0)

    def issue(tab_hbm, b, buf, sm):
        pltpu.async_copy(tab_hbm.at[ctx_v.at[b, pl.ds(0, C1)]],
                         buf.at[pl.ds(0, C1)], sm)
        pltpu.async_copy(tab_hbm.at[ctx_v.at[b, pl.ds(C1, C2)]],
                         buf.at[pl.ds(C1, C2)], sm)

    def drain(tab_hbm, buf, sm):
        pltpu.make_async_copy(tab_hbm.at[ctx_v.at[0, pl.ds(0, C1)]],
                              buf.at[pl.ds(0, C1)], sm).wait()
        pltpu.make_async_copy(tab_hbm.at[ctx_v.at[0, pl.ds(C1, C2)]],
                              buf.at[pl.ds(C1, C2)], sm).wait()

    def accum(b, buf):
        def row_body(r, carry):
            a0, a1 = carry
            l0 = r * 8
            for u in range(8):
                a0 = a0 + buf[l0 + u, pl.ds(0, HALF)]
                a1 = a1 + buf[l0 + u, pl.ds(HALF, HALF)]
            return a0, a1

        a0, a1 = lax.fori_loop(0, L // 8, row_body, (zero, zero))
        p = (a0 * nrows_v[b, pl.ds(0, HALF)]
             + a1 * nrows_v[b, pl.ds(HALF, HALF)])
        off = b * HALF
        pacc_v[pl.ds(off, HALF)] = pacc_v[pl.ds(off, HALF)] + p

    pltpu.sync_copy(ctx_hbm.at[pl.ds(base, BPW)], ctx_v)
    pltpu.sync_copy(next_hbm.at[pl.ds(base, BPW)], nidx_v)
    pltpu.async_copy(tab_hbm.at[nidx_v], nrows_v, nsem).wait()

    for u in range(NBUF - 1):
        issue(tab_hbm, u, bufs[u], sems[u])

    def g_body(g, _):
        for u in range(NBUF):
            b = NBUF * g + u
            b_next = b + NBUF - 1

            @pl.when(b_next < BPW)
            def _():
                issue(tab_hbm, b_next, bufs[(u + NBUF - 1) % NBUF],
                      sems[(u + NBUF - 1) % NBUF])

            drain(tab_hbm, bufs[u], sems[u])
            accum(b, bufs[u])
        return 0

    lax.fori_loop(0, BPW // NBUF, g_body, 0)

    inv_l = jnp.float32(1.0 / L)

    def out_chunk(g, _):
        def out_lane(j, out_acc):
            p = pacc_v[pl.ds((g * HALF + j) * HALF, HALF)]
            for sh in (8, 4, 2, 1):
                p = p + lax.gather(
                    p, (lanes ^ sh)[:, None], _GATHER_DNUMS, (1,),
                    mode=lax.GatherScatterMode.PROMISE_IN_BOUNDS)
            return jnp.where(lanes == j, p, out_acc)

        out_acc = lax.fori_loop(0, HALF, out_lane, zero)
        out_v[pl.ds(g * HALF, HALF)] = out_acc * inv_l
        return 0

    lax.fori_loop(0, BPW // HALF, out_chunk, 0)
    pltpu.sync_copy(out_v, out_hbm.at[pl.ds(base, BPW)])


_spotify_sc = functools.partial(
    pl.kernel,
    mesh=plsc.VectorSubcoreMesh(core_axis_name="c", subcore_axis_name="s"),
    out_type=jax.ShapeDtypeStruct((B,), jnp.float32),
    compiler_params=pltpu.CompilerParams(use_tc_tiling_on_sc=False),
    scratch_types=[
        pltpu.VMEM((BPW, L), jnp.int32),
        pltpu.VMEM((BPW,), jnp.int32),
        pltpu.VMEM((L, F), jnp.float32),
        pltpu.VMEM((L, F), jnp.float32),
        pltpu.VMEM((L, F), jnp.float32),
        pltpu.VMEM((L, F), jnp.float32),
        pltpu.VMEM((BPW, F), jnp.float32),
        pltpu.VMEM((BPW * HALF,), jnp.float32),
        pltpu.VMEM((BPW,), jnp.float32),
        pltpu.SemaphoreType.DMA,
        pltpu.SemaphoreType.DMA,
        pltpu.SemaphoreType.DMA,
        pltpu.SemaphoreType.DMA,
        pltpu.SemaphoreType.DMA,
    ],
)(_sc_body)


def kernel(track_context, artist_context, album_context,
           next_track, next_artist, next_album,
           track_table, artist_table, album_table):
    tc = _perm_ids(track_context)
    ac = _perm_ids(artist_context)
    alc = _perm_ids(album_context)
    nt = _perm_ids(next_track.reshape(-1))
    na = _perm_ids(next_artist.reshape(-1))
    nal = _perm_ids(next_album.reshape(-1))
    t0 = _linearize_table(track_table)
    t1 = _linearize_table(artist_table)
    t2 = _linearize_table(album_table)
    o0 = _spotify_sc(tc, nt, t0)
    o1 = _spotify_sc(ac, na, t1)
    o2 = _spotify_sc(alc, nal, t2)
    return o0 + o1 + o2


# speedup vs baseline: 3.9520x; 1.0263x over previous
import functools

import jax
import jax.numpy as jnp
from jax import lax
from jax.experimental import pallas as pl
from jax.experimental.pallas import tpu as pltpu
from jax.experimental.pallas import tpu_sc as plsc

B = 4096
L = 200
F = 32
NC = 2
NS = 16
NW = NC * NS
BPW = B // NW
C1 = 128
C2 = L - C1
HALF = F // 2
NBUF = 8
CH = 16384

_GATHER_DNUMS = lax.GatherDimensionNumbers(
    offset_dims=(), collapsed_slice_dims=(0,), start_index_map=(0,))


QT = CH // 4


def _tp_body(x_ref, o_ref):
    xs = jnp.concatenate(
        [x_ref[:, a * QT:(a + 1) * QT] for a in range(4)], axis=0)
    o_ref[...] = xs.T


def _linearize_table(table):
    v = table.shape[0]
    tab_t = table.T
    nblk = (v + CH - 1) // CH
    t4 = pl.pallas_call(
        _tp_body,
        grid=(nblk,),
        in_specs=[pl.BlockSpec((F, CH), lambda c: (0, c))],
        out_specs=pl.BlockSpec((QT, 128), lambda c: (c, 0)),
        out_shape=jax.ShapeDtypeStruct((nblk * QT, 128), jnp.float32),
        compiler_params=pltpu.CompilerParams(fuse_transposed_lhs_in_matmul=True),
    )(tab_t)
    return t4.reshape(nblk * CH, F)


def _perm_ids(ids):
    i = ids.astype(jnp.int32)
    c = i // CH
    l = i % CH
    return (c * QT + l % QT) * 4 + l // QT


def _sc_body(ctx_hbm, next_hbm, tab_hbm, out_hbm,
             ctx_v, nidx_v, r0_v, r1_v, r2_v, r3_v, r4_v, r5_v, r6_v, r7_v,
             nrows_v, pacc_v, out_v,
             sem0, sem1, sem2, sem3, sem4, sem5, sem6, sem7, nsem):
    wid = lax.axis_index("s") * NC + lax.axis_index("c")
    base = wid * BPW

    zero = jnp.zeros((HALF,), jnp.float32)
    lanes = lax.iota(jnp.int32, HALF)
    bufs = (r0_v, r1_v, r2_v, r3_v, r4_v, r5_v, r6_v, r7_v)
    sems = (sem0, sem1, sem2, sem3, sem4, sem5, sem6, sem7)

    def zpacc(i, _):
        pacc_v[pl.ds(i * HALF, HALF)] = zero
        return 0

    lax.fori_loop(0, BPW, zpacc, 0)

    def issue(tab_hbm, b, buf, sm):
        pltpu.async_copy(tab_hbm.at[ctx_v.at[b, pl.ds(0, C1)]],
                         buf.at[pl.ds(0, C1)], sm)
        pltpu.async_copy(tab_hbm.at[ctx_v.at[b, pl.ds(C1, C2)]],
                         buf.at[pl.ds(C1, C2)], sm)

    def drain(tab_hbm, buf, sm):
        pltpu.make_async_copy(tab_hbm.at[ctx_v.at[0, pl.ds(0, C1)]],
                              buf.at[pl.ds(0, C1)], sm).wait()
        pltpu.make_async_copy(tab_hbm.at[ctx_v.at[0, pl.ds(C1, C2)]],
                              buf.at[pl.ds(C1, C2)], sm).wait()

    def accum(b, buf):
        def row_body(r, carry):
            a0, a1 = carry
            l0 = r * 8
            for u in range(8):
                a0 = a0 + buf[l0 + u, pl.ds(0, HALF)]
                a1 = a1 + buf[l0 + u, pl.ds(HALF, HALF)]
            return a0, a1

        a0, a1 = lax.fori_loop(0, L // 8, row_body, (zero, zero))
        p = (a0 * nrows_v[b, pl.ds(0, HALF)]
             + a1 * nrows_v[b, pl.ds(HALF, HALF)])
        off = b * HALF
        pacc_v[pl.ds(off, HALF)] = pacc_v[pl.ds(off, HALF)] + p

    pltpu.sync_copy(ctx_hbm.at[pl.ds(base, BPW)], ctx_v)
    pltpu.sync_copy(next_hbm.at[pl.ds(base, BPW)], nidx_v)
    pltpu.async_copy(tab_hbm.at[nidx_v], nrows_v, nsem).wait()

    for u in range(NBUF - 1):
        issue(tab_hbm, u, bufs[u], sems[u])

    def g_body(g, _):
        for u in range(NBUF):
            b = NBUF * g + u
            b_next = b + NBUF - 1

            @pl.when(b_next < BPW)
            def _():
                issue(tab_hbm, b_next, bufs[(u + NBUF - 1) % NBUF],
                      sems[(u + NBUF - 1) % NBUF])

            drain(tab_hbm, bufs[u], sems[u])
            accum(b, bufs[u])
        return 0

    lax.fori_loop(0, BPW // NBUF, g_body, 0)

    inv_l = jnp.float32(1.0 / L)

    def out_chunk(g, _):
        def out_lane(j, out_acc):
            p = pacc_v[pl.ds((g * HALF + j) * HALF, HALF)]
            for sh in (8, 4, 2, 1):
                p = p + lax.gather(
                    p, (lanes ^ sh)[:, None], _GATHER_DNUMS, (1,),
                    mode=lax.GatherScatterMode.PROMISE_IN_BOUNDS)
            return jnp.where(lanes == j, p, out_acc)

        out_acc = lax.fori_loop(0, HALF, out_lane, zero)
        out_v[pl.ds(g * HALF, HALF)] = out_acc * inv_l
        return 0

    lax.fori_loop(0, BPW // HALF, out_chunk, 0)
    pltpu.sync_copy(out_v, out_hbm.at[pl.ds(base, BPW)])


_spotify_sc = functools.partial(
    pl.kernel,
    mesh=plsc.VectorSubcoreMesh(core_axis_name="c", subcore_axis_name="s"),
    out_type=jax.ShapeDtypeStruct((B,), jnp.float32),
    compiler_params=pltpu.CompilerParams(use_tc_tiling_on_sc=False),
    scratch_types=[
        pltpu.VMEM((BPW, L), jnp.int32),
        pltpu.VMEM((BPW,), jnp.int32),
        pltpu.VMEM((L, F), jnp.float32),
        pltpu.VMEM((L, F), jnp.float32),
        pltpu.VMEM((L, F), jnp.float32),
        pltpu.VMEM((L, F), jnp.float32),
        pltpu.VMEM((L, F), jnp.float32),
        pltpu.VMEM((L, F), jnp.float32),
        pltpu.VMEM((L, F), jnp.float32),
        pltpu.VMEM((L, F), jnp.float32),
        pltpu.VMEM((BPW, F), jnp.float32),
        pltpu.VMEM((BPW * HALF,), jnp.float32),
        pltpu.VMEM((BPW,), jnp.float32),
        pltpu.SemaphoreType.DMA,
        pltpu.SemaphoreType.DMA,
        pltpu.SemaphoreType.DMA,
        pltpu.SemaphoreType.DMA,
        pltpu.SemaphoreType.DMA,
        pltpu.SemaphoreType.DMA,
        pltpu.SemaphoreType.DMA,
        pltpu.SemaphoreType.DMA,
        pltpu.SemaphoreType.DMA,
    ],
)(_sc_body)


def kernel(track_context, artist_context, album_context,
           next_track, next_artist, next_album,
           track_table, artist_table, album_table):
    tc = _perm_ids(track_context)
    ac = _perm_ids(artist_context)
    alc = _perm_ids(album_context)
    nt = _perm_ids(next_track.reshape(-1))
    na = _perm_ids(next_artist.reshape(-1))
    nal = _perm_ids(next_album.reshape(-1))
    t0 = _linearize_table(track_table)
    t1 = _linearize_table(artist_table)
    t2 = _linearize_table(album_table)
    o1 = _spotify_sc(ac, na, t1)
    o2 = _spotify_sc(alc, nal, t2)
    o0 = _spotify_sc(tc, nt, t0)
    return o0 + o1 + o2


# speedup vs baseline: 3.9524x; 1.0001x over previous
import functools

import jax
import jax.numpy as jnp
from jax import lax
from jax.experimental import pallas as pl
from jax.experimental.pallas import tpu as pltpu
from jax.experimental.pallas import tpu_sc as plsc

B = 4096
L = 200
F = 32
NC = 2
NS = 16
NW = NC * NS
BPW = B // NW
C1 = 128
C2 = L - C1
HALF = F // 2
NBUF = 8
CH = 16384

_GATHER_DNUMS = lax.GatherDimensionNumbers(
    offset_dims=(), collapsed_slice_dims=(0,), start_index_map=(0,))


QT = CH // 4


def _tp_body(x_ref, o_ref):
    xs = jnp.concatenate(
        [x_ref[:, a * QT:(a + 1) * QT] for a in range(4)], axis=0)
    o_ref[...] = xs.T


def _linearize_table(table):
    v = table.shape[0]
    tab_t = table.T
    nblk = (v + CH - 1) // CH
    t4 = pl.pallas_call(
        _tp_body,
        grid=(nblk,),
        in_specs=[pl.BlockSpec((F, CH), lambda c: (0, c))],
        out_specs=pl.BlockSpec((QT, 128), lambda c: (c, 0)),
        out_shape=jax.ShapeDtypeStruct((nblk * QT, 128), jnp.float32),
    )(tab_t)
    return t4.reshape(nblk * CH, F)


def _perm_ids(ids):
    i = ids.astype(jnp.int32)
    c = i // CH
    l = i % CH
    return (c * QT + l % QT) * 4 + l // QT


def _sc_body(ctx_hbm, next_hbm, tab_hbm, out_hbm,
             ctx_v, nidx_v, r0_v, r1_v, r2_v, r3_v, r4_v, r5_v, r6_v, r7_v,
             nrows_v, pacc_v, out_v,
             sem0, sem1, sem2, sem3, sem4, sem5, sem6, sem7, nsem):
    wid = lax.axis_index("s") * NC + lax.axis_index("c")
    base = wid * BPW

    zero = jnp.zeros((HALF,), jnp.float32)
    lanes = lax.iota(jnp.int32, HALF)
    bufs = (r0_v, r1_v, r2_v, r3_v, r4_v, r5_v, r6_v, r7_v)
    sems = (sem0, sem1, sem2, sem3, sem4, sem5, sem6, sem7)

    def zpacc(i, _):
        pacc_v[pl.ds(i * HALF, HALF)] = zero
        return 0

    lax.fori_loop(0, BPW, zpacc, 0)

    def issue(tab_hbm, b, buf, sm):
        pltpu.async_copy(tab_hbm.at[ctx_v.at[b, pl.ds(0, C1)]],
                         buf.at[pl.ds(0, C1)], sm)
        pltpu.async_copy(tab_hbm.at[ctx_v.at[b, pl.ds(C1, C2)]],
                         buf.at[pl.ds(C1, C2)], sm)

    def drain(tab_hbm, buf, sm):
        pltpu.make_async_copy(tab_hbm.at[ctx_v.at[0, pl.ds(0, C1)]],
                              buf.at[pl.ds(0, C1)], sm).wait()
        pltpu.make_async_copy(tab_hbm.at[ctx_v.at[0, pl.ds(C1, C2)]],
                              buf.at[pl.ds(C1, C2)], sm).wait()

    def accum(b, buf):
        def row_body(r, carry):
            a0, a1 = carry
            l0 = r * 8
            for u in range(8):
                a0 = a0 + buf[l0 + u, pl.ds(0, HALF)]
                a1 = a1 + buf[l0 + u, pl.ds(HALF, HALF)]
            return a0, a1

        a0, a1 = lax.fori_loop(0, L // 8, row_body, (zero, zero))
        p = (a0 * nrows_v[b, pl.ds(0, HALF)]
             + a1 * nrows_v[b, pl.ds(HALF, HALF)])
        off = b * HALF
        pacc_v[pl.ds(off, HALF)] = pacc_v[pl.ds(off, HALF)] + p

    pltpu.sync_copy(ctx_hbm.at[pl.ds(base, BPW)], ctx_v)
    pltpu.sync_copy(next_hbm.at[pl.ds(base, BPW)], nidx_v)
    pltpu.async_copy(tab_hbm.at[nidx_v], nrows_v, nsem).wait()

    for u in range(NBUF - 1):
        issue(tab_hbm, u, bufs[u], sems[u])

    def g_body(g, _):
        for u in range(NBUF):
            b = NBUF * g + u
            b_next = b + NBUF - 1

            @pl.when(b_next < BPW)
            def _():
                issue(tab_hbm, b_next, bufs[(u + NBUF - 1) % NBUF],
                      sems[(u + NBUF - 1) % NBUF])

            drain(tab_hbm, bufs[u], sems[u])
            accum(b, bufs[u])
        return 0

    lax.fori_loop(0, BPW // NBUF, g_body, 0)

    inv_l = jnp.float32(1.0 / L)

    def out_chunk(g, _):
        def out_lane(j, out_acc):
            p = pacc_v[pl.ds((g * HALF + j) * HALF, HALF)]
            for sh in (8, 4, 2, 1):
                p = p + lax.gather(
                    p, (lanes ^ sh)[:, None], _GATHER_DNUMS, (1,),
                    mode=lax.GatherScatterMode.PROMISE_IN_BOUNDS)
            return jnp.where(lanes == j, p, out_acc)

        out_acc = lax.fori_loop(0, HALF, out_lane, zero)
        out_v[pl.ds(g * HALF, HALF)] = out_acc * inv_l
        return 0

    lax.fori_loop(0, BPW // HALF, out_chunk, 0)
    pltpu.sync_copy(out_v, out_hbm.at[pl.ds(base, BPW)])


_spotify_sc = functools.partial(
    pl.kernel,
    mesh=plsc.VectorSubcoreMesh(core_axis_name="c", subcore_axis_name="s"),
    out_type=jax.ShapeDtypeStruct((B,), jnp.float32),
    compiler_params=pltpu.CompilerParams(use_tc_tiling_on_sc=False),
    scratch_types=[
        pltpu.VMEM((BPW, L), jnp.int32),
        pltpu.VMEM((BPW,), jnp.int32),
        pltpu.VMEM((L, F), jnp.float32),
        pltpu.VMEM((L, F), jnp.float32),
        pltpu.VMEM((L, F), jnp.float32),
        pltpu.VMEM((L, F), jnp.float32),
        pltpu.VMEM((L, F), jnp.float32),
        pltpu.VMEM((L, F), jnp.float32),
        pltpu.VMEM((L, F), jnp.float32),
        pltpu.VMEM((L, F), jnp.float32),
        pltpu.VMEM((BPW, F), jnp.float32),
        pltpu.VMEM((BPW * HALF,), jnp.float32),
        pltpu.VMEM((BPW,), jnp.float32),
        pltpu.SemaphoreType.DMA,
        pltpu.SemaphoreType.DMA,
        pltpu.SemaphoreType.DMA,
        pltpu.SemaphoreType.DMA,
        pltpu.SemaphoreType.DMA,
        pltpu.SemaphoreType.DMA,
        pltpu.SemaphoreType.DMA,
        pltpu.SemaphoreType.DMA,
        pltpu.SemaphoreType.DMA,
    ],
)(_sc_body)


def kernel(track_context, artist_context, album_context,
           next_track, next_artist, next_album,
           track_table, artist_table, album_table):
    tc = _perm_ids(track_context)
    ac = _perm_ids(artist_context)
    alc = _perm_ids(album_context)
    nt = _perm_ids(next_track.reshape(-1))
    na = _perm_ids(next_artist.reshape(-1))
    nal = _perm_ids(next_album.reshape(-1))
    t0 = _linearize_table(track_table)
    t1 = _linearize_table(artist_table)
    t2 = _linearize_table(album_table)
    o1 = _spotify_sc(ac, na, t1)
    o2 = _spotify_sc(alc, nal, t2)
    o0 = _spotify_sc(tc, nt, t0)
    return o0 + o1 + o2
